# baseline (speedup 1.0000x reference)
import functools

import jax
import jax.numpy as jnp
from jax import lax
from jax.experimental import pallas as pl
from jax.experimental.pallas import tpu as pltpu
from jax.experimental.pallas import tpu_sc as plsc

_B = 16384
_SEQ = 10
_EMBED = 16
_HIDDEN = 32
_FEAT = _SEQ * _EMBED
_V = 1000000

_NC, _NS = 2, 16
_NW = _NC * _NS
_N = _B * _SEQ
_CHUNK = 128
_NCHUNK = _N // _NW // _CHUNK
_PER_W = _NCHUNK * _CHUNK
_BW = _B // _NW
_NG = _BW // 16


def _fused(x_chunks, table_lin, W1, b1, W2f, b2):
    mesh = plsc.VectorSubcoreMesh(
        core_axis_name="c", subcore_axis_name="s",
        num_cores=_NC, num_subcores=_NS)

    @functools.partial(
        pl.kernel,
        out_type=jax.ShapeDtypeStruct((_B,), jnp.float32),
        mesh=mesh,
        scratch_types=[
            pltpu.VMEM((_NCHUNK, _CHUNK), jnp.int32),
            pltpu.VMEM((_EMBED, _PER_W + 16), jnp.float32),
            pltpu.VMEM((2 * _FEAT, 16), jnp.float32),
            pltpu.VMEM((_HIDDEN,), jnp.float32),
            pltpu.VMEM((_HIDDEN,), jnp.float32),
            pltpu.VMEM((16,), jnp.float32),
            pltpu.VMEM((_BW,), jnp.float32),
            pltpu.SemaphoreType.DMA,
            pltpu.SemaphoreType.DMA,
        ],
        compiler_params=pltpu.CompilerParams(
            use_tc_tiling_on_sc=False, needs_layout_passes=False),
    )
    def fused_kernel(x_hbm, tab_hbm, w1_hbm, b1_hbm, w2_hbm, b2_hbm,
                     out_hbm, idx_v, rows_v, w1_v, b1_v, w2_v, b2_v, out_v,
                     sem, wsem):
        wid = lax.axis_index("s") * _NC + lax.axis_index("c")

        pltpu.async_copy(w1_hbm, w1_v, wsem)
        pltpu.async_copy(b1_hbm, b1_v, wsem)
        pltpu.async_copy(w2_hbm, w2_v, wsem)
        pltpu.async_copy(b2_hbm, b2_v, wsem)
        pltpu.sync_copy(x_hbm.at[wid], idx_v)
        pltpu.make_async_copy(w1_hbm, w1_v, wsem).wait()
        pltpu.make_async_copy(b1_hbm, b1_v, wsem).wait()
        pltpu.make_async_copy(w2_hbm, w2_v, wsem).wait()
        pltpu.make_async_copy(b2_hbm, b2_v, wsem).wait()

        @pl.loop(0, _NCHUNK)
        def _fire(j):
            for d in range(_EMBED):
                pltpu.async_copy(
                    tab_hbm.at[pl.ds(d * _V, _V)].at[idx_v.at[j]],
                    rows_v.at[d, pl.ds(j * _CHUNK, _CHUNK)], sem)

        @pl.loop(0, _NCHUNK)
        def _drain(j):
            for d in range(_EMBED):
                pltpu.make_async_copy(
                    tab_hbm.at[pl.ds(d * _V, _V)].at[idx_v.at[j]],
                    rows_v.at[d, pl.ds(j * _CHUNK, _CHUNK)], sem).wait()

        lane = lax.iota(jnp.int32, 16)
        b1a = b1_v[pl.ds(0, 16)]
        b1b = b1_v[pl.ds(16, 16)]
        w2a = w2_v[pl.ds(0, 16)]
        w2b = w2_v[pl.ds(16, 16)]
        b2vec = b2_v[...]

        @pl.loop(0, _NG)
        def _group(bb):
            def s_body(s, h):
                h = list(h)
                base = bb * (16 * _SEQ) + s
                for d in range(_EMBED):
                    k2 = 2 * (s * _EMBED + d)
                    w1a = w1_v[k2]
                    w1b = w1_v[k2 + 1]
                    for i in range(16):
                        ev = rows_v[d, pl.ds(base + i * _SEQ, 16)]
                        e = ev[0]
                        h[2 * i] = h[2 * i] + e * w1a
                        h[2 * i + 1] = h[2 * i + 1] + e * w1b
                return tuple(h)

            h0 = tuple(
                jnp.full((16,), 0.0, jnp.float32) for _ in range(_HIDDEN))
            h = lax.fori_loop(0, _SEQ, s_body, h0)

            o = jnp.full((16,), 0.0, jnp.float32)
            for i in range(16):
                ta = jnp.maximum(h[2 * i] + b1a, 0.0) * w2a
                tb = jnp.maximum(h[2 * i + 1] + b1b, 0.0) * w2b
                s_i = jnp.sum(ta + tb)
                o = jnp.where(lane == i, o + s_i, o)
            o = o + b2vec
            out_v[pl.ds(bb * 16, 16)] = 1.0 / (1.0 + jnp.exp(-o))

        pltpu.sync_copy(out_v, out_hbm.at[pl.ds(wid * _BW, _BW)])

    return fused_kernel(x_chunks, table_lin, W1, b1, W2f, b2)


def kernel(x, table, W1, b1, W2, b2):
    x_chunks = x.astype(jnp.int32).reshape(_NW, _NCHUNK, _CHUNK)
    table_lin = table.T.reshape(_EMBED * _V)
    w1r = W1.reshape(2 * _FEAT, 16)
    b2vec = jnp.full((16,), b2[0], jnp.float32)
    out = _fused(x_chunks, table_lin, w1r, b1, W2.reshape(_HIDDEN), b2vec)
    return out.reshape(_B, 1)


# speedup vs baseline: 5.8946x; 5.8946x over previous
import functools

import jax
import jax.numpy as jnp
from jax import lax
from jax.experimental import pallas as pl
from jax.experimental.pallas import tpu as pltpu
from jax.experimental.pallas import tpu_sc as plsc

_B = 16384
_SEQ = 10
_EMBED = 16
_HIDDEN = 32
_FEAT = _SEQ * _EMBED
_V = 1000000

_NC, _NS = 2, 16
_NW = _NC * _NS
_TLEN = 16252928
_N = _B * _SEQ
_CHUNK = 128
_NCHUNK = _N // _NW // _CHUNK
_PER_W = _NCHUNK * _CHUNK
_BW = _B // _NW
_NG = _BW // 16


def _fused(x_chunks, table_lin, W1, b1, W2f, b2):
    mesh = plsc.VectorSubcoreMesh(
        core_axis_name="c", subcore_axis_name="s",
        num_cores=_NC, num_subcores=_NS)

    @functools.partial(
        pl.kernel,
        out_type=jax.ShapeDtypeStruct((_B,), jnp.float32),
        mesh=mesh,
        scratch_types=[
            pltpu.VMEM((_NCHUNK, _CHUNK), jnp.int32),
            pltpu.VMEM((_NCHUNK, _CHUNK), jnp.int32),
            pltpu.VMEM((_EMBED, _PER_W + 16), jnp.float32),
            pltpu.VMEM((2 * _FEAT, 16), jnp.float32),
            pltpu.VMEM((_HIDDEN,), jnp.float32),
            pltpu.VMEM((_HIDDEN,), jnp.float32),
            pltpu.VMEM((16,), jnp.float32),
            pltpu.VMEM((_BW,), jnp.float32),
            pltpu.SemaphoreType.DMA,
            pltpu.SemaphoreType.DMA,
        ],
        compiler_params=pltpu.CompilerParams(
            use_tc_tiling_on_sc=False, needs_layout_passes=False),
    )
    def fused_kernel(x_hbm, tab_hbm, w1_hbm, b1_hbm, w2_hbm, b2_hbm,
                     out_hbm, idx_v, idxb_v, rows_v, w1_v, b1_v, w2_v, b2_v,
                     out_v, sem, wsem):
        wid = lax.axis_index("s") * _NC + lax.axis_index("c")

        pltpu.async_copy(w1_hbm, w1_v, wsem)
        pltpu.async_copy(b1_hbm, b1_v, wsem)
        pltpu.async_copy(w2_hbm, w2_v, wsem)
        pltpu.async_copy(b2_hbm, b2_v, wsem)
        pltpu.sync_copy(x_hbm.at[wid], idx_v)
        pltpu.make_async_copy(w1_hbm, w1_v, wsem).wait()
        pltpu.make_async_copy(b1_hbm, b1_v, wsem).wait()
        pltpu.make_async_copy(w2_hbm, w2_v, wsem).wait()
        pltpu.make_async_copy(b2_hbm, b2_v, wsem).wait()

        @pl.loop(0, _NCHUNK)
        def _prep(j):
            for m in range(_CHUNK // 16):
                v = idx_v[j, pl.ds(m * 16, 16)]
                idxb_v[j, pl.ds(m * 16, 16)] = (
                    ((v >> 7) << 11) | (v & 127))

        @pl.loop(0, _NCHUNK)
        def _fire(j):
            for d in range(_EMBED):
                pltpu.async_copy(
                    tab_hbm.at[pl.ds(d * _CHUNK, _TLEN - d * _CHUNK)]
                           .at[idxb_v.at[j]],
                    rows_v.at[d, pl.ds(j * _CHUNK, _CHUNK)], sem)

        @pl.loop(0, _NCHUNK)
        def _drain(j):
            for d in range(_EMBED):
                pltpu.make_async_copy(
                    tab_hbm.at[pl.ds(d * _CHUNK, _TLEN - d * _CHUNK)]
                           .at[idxb_v.at[j]],
                    rows_v.at[d, pl.ds(j * _CHUNK, _CHUNK)], sem).wait()

        lane = lax.iota(jnp.int32, 16)
        b1a = b1_v[pl.ds(0, 16)]
        b1b = b1_v[pl.ds(16, 16)]
        w2a = w2_v[pl.ds(0, 16)]
        w2b = w2_v[pl.ds(16, 16)]
        b2vec = b2_v[...]

        @pl.loop(0, _NG)
        def _group(bb):
            def s_body(s, h):
                h = list(h)
                base = bb * (16 * _SEQ) + s
                for d in range(_EMBED):
                    k2 = 2 * (s * _EMBED + d)
                    w1a = w1_v[k2]
                    w1b = w1_v[k2 + 1]
                    for i in range(16):
                        ev = rows_v[d, pl.ds(base + i * _SEQ, 16)]
                        e = ev[0]
                        h[2 * i] = h[2 * i] + e * w1a
                        h[2 * i + 1] = h[2 * i + 1] + e * w1b
                return tuple(h)

            h0 = tuple(
                jnp.full((16,), 0.0, jnp.float32) for _ in range(_HIDDEN))
            h = lax.fori_loop(0, _SEQ, s_body, h0)

            o = jnp.full((16,), 0.0, jnp.float32)
            for i in range(16):
                ta = jnp.maximum(h[2 * i] + b1a, 0.0) * w2a
                tb = jnp.maximum(h[2 * i + 1] + b1b, 0.0) * w2b
                s_i = jnp.sum(ta + tb)
                o = jnp.where(lane == i, o + s_i, o)
            o = o + b2vec
            out_v[pl.ds(bb * 16, 16)] = 1.0 / (1.0 + jnp.exp(-o))

        pltpu.sync_copy(out_v, out_hbm.at[pl.ds(wid * _BW, _BW)])

    return fused_kernel(x_chunks, table_lin, W1, b1, W2f, b2)


_VBLK = 32768
_NBLK = (_V + _VBLK - 1) // _VBLK


def _detile_body(t_ref, o_ref):
    a3 = t_ref[...].reshape(_EMBED, _VBLK // 128, 128)
    o_ref[...] = a3.transpose(1, 0, 2).reshape(_VBLK * _EMBED // 128, 128)


def _tc_detile(tableT):
    return pl.pallas_call(
        _detile_body,
        grid=(_NBLK,),
        in_specs=[pl.BlockSpec((_EMBED, _VBLK), lambda i: (0, i))],
        out_specs=pl.BlockSpec((_VBLK * _EMBED // 128, 128), lambda i: (i, 0)),
        out_shape=jax.ShapeDtypeStruct(
            (_NBLK * _VBLK * _EMBED // 128, 128), jnp.float32),
    )(tableT)


def kernel(x, table, W1, b1, W2, b2):
    x_chunks = x.astype(jnp.int32).reshape(_NW, _NCHUNK, _CHUNK)
    tab_flat = _tc_detile(table.T).reshape(_TLEN)
    w1r = W1.reshape(2 * _FEAT, 16)
    b2vec = jnp.full((16,), b2[0], jnp.float32)
    out = _fused(x_chunks, tab_flat, w1r, b1, W2.reshape(_HIDDEN), b2vec)
    return out.reshape(_B, 1)


# speedup vs baseline: 5.9894x; 1.0161x over previous
import functools

import jax
import jax.numpy as jnp
from jax import lax
from jax.experimental import pallas as pl
from jax.experimental.pallas import tpu as pltpu
from jax.experimental.pallas import tpu_sc as plsc

_B = 16384
_SEQ = 10
_EMBED = 16
_HIDDEN = 32
_FEAT = _SEQ * _EMBED
_V = 1000000

_NC, _NS = 2, 16
_NW = _NC * _NS
_TLEN = 16252928
_N = _B * _SEQ
_CHUNK = 128
_NCHUNK = _N // _NW // _CHUNK
_PER_W = _NCHUNK * _CHUNK
_BW = _B // _NW
_NG = _BW // 16


def _fused(x_chunks, table_lin, W1, b1, W2f, b2):
    mesh = plsc.VectorSubcoreMesh(
        core_axis_name="c", subcore_axis_name="s",
        num_cores=_NC, num_subcores=_NS)

    @functools.partial(
        pl.kernel,
        out_type=jax.ShapeDtypeStruct((_B,), jnp.float32),
        mesh=mesh,
        scratch_types=[
            pltpu.VMEM((_NCHUNK, _CHUNK), jnp.int32),
            pltpu.VMEM((_NCHUNK, _CHUNK), jnp.int32),
            pltpu.VMEM((_EMBED, _PER_W + 16), jnp.float32),
            pltpu.VMEM((2 * _FEAT, 16), jnp.float32),
            pltpu.VMEM((_HIDDEN,), jnp.float32),
            pltpu.VMEM((_HIDDEN,), jnp.float32),
            pltpu.VMEM((16,), jnp.float32),
            pltpu.VMEM((_BW,), jnp.float32),
            pltpu.SemaphoreType.DMA,
            pltpu.SemaphoreType.DMA,
        ],
        compiler_params=pltpu.CompilerParams(
            use_tc_tiling_on_sc=False, needs_layout_passes=False),
    )
    def fused_kernel(x_hbm, tab_hbm, w1_hbm, b1_hbm, w2_hbm, b2_hbm,
                     out_hbm, idx_v, idxb_v, rows_v, w1_v, b1_v, w2_v, b2_v,
                     out_v, sem, wsem):
        wid = lax.axis_index("s") * _NC + lax.axis_index("c")

        pltpu.async_copy(w1_hbm, w1_v, wsem)
        pltpu.async_copy(b1_hbm, b1_v, wsem)
        pltpu.async_copy(w2_hbm, w2_v, wsem)
        pltpu.async_copy(b2_hbm, b2_v, wsem)
        pltpu.sync_copy(x_hbm.at[wid], idx_v)
        pltpu.make_async_copy(w1_hbm, w1_v, wsem).wait()
        pltpu.make_async_copy(b1_hbm, b1_v, wsem).wait()
        pltpu.make_async_copy(w2_hbm, w2_v, wsem).wait()
        pltpu.make_async_copy(b2_hbm, b2_v, wsem).wait()

        @pl.loop(0, _NCHUNK)
        def _prep(j):
            for m in range(_CHUNK // 16):
                v = idx_v[j, pl.ds(m * 16, 16)]
                idxb_v[j, pl.ds(m * 16, 16)] = (
                    ((v >> 7) << 11) | (v & 127))

        @pl.loop(0, _NCHUNK)
        def _fire(j):
            for d in range(_EMBED):
                pltpu.async_copy(
                    tab_hbm.at[pl.ds(d * _CHUNK, _TLEN - d * _CHUNK)]
                           .at[idxb_v.at[j]],
                    rows_v.at[d, pl.ds(j * _CHUNK, _CHUNK)], sem)

        lane = lax.iota(jnp.int32, 16)
        b1a = b1_v[pl.ds(0, 16)]
        b1b = b1_v[pl.ds(16, 16)]
        w2a = w2_v[pl.ds(0, 16)]
        w2b = w2_v[pl.ds(16, 16)]
        b2vec = b2_v[...]

        @pl.loop(0, 4)
        def _piece(p):
            @pl.loop(0, _NCHUNK // 4)
            def _drain(c):
                j = p * (_NCHUNK // 4) + c
                for d in range(_EMBED):
                    pltpu.make_async_copy(
                        tab_hbm.at[pl.ds(d * _CHUNK, _TLEN - d * _CHUNK)]
                               .at[idxb_v.at[j]],
                        rows_v.at[d, pl.ds(j * _CHUNK, _CHUNK)], sem).wait()

            @pl.loop(0, _NG // 4)
            def _group(g):
                bb = p * (_NG // 4) + g

                def s_body(s, h):
                    h = list(h)
                    base = bb * (16 * _SEQ) + s
                    for d in range(_EMBED):
                        k2 = 2 * (s * _EMBED + d)
                        w1a = w1_v[k2]
                        w1b = w1_v[k2 + 1]
                        for i in range(16):
                            ev = rows_v[d, pl.ds(base + i * _SEQ, 16)]
                            e = ev[0]
                            h[2 * i] = h[2 * i] + e * w1a
                            h[2 * i + 1] = h[2 * i + 1] + e * w1b
                    return tuple(h)

                h0 = tuple(
                    jnp.full((16,), 0.0, jnp.float32) for _ in range(_HIDDEN))
                h = lax.fori_loop(0, _SEQ, s_body, h0)

                o = jnp.full((16,), 0.0, jnp.float32)
                for i in range(16):
                    ta = jnp.maximum(h[2 * i] + b1a, 0.0) * w2a
                    tb = jnp.maximum(h[2 * i + 1] + b1b, 0.0) * w2b
                    s_i = jnp.sum(ta + tb)
                    o = jnp.where(lane == i, o + s_i, o)
                o = o + b2vec
                out_v[pl.ds(bb * 16, 16)] = 1.0 / (1.0 + jnp.exp(-o))

        pltpu.sync_copy(out_v, out_hbm.at[pl.ds(wid * _BW, _BW)])

    return fused_kernel(x_chunks, table_lin, W1, b1, W2f, b2)


_VBLK = 32768
_NBLK = (_V + _VBLK - 1) // _VBLK


def _detile_body(t_ref, o_ref):
    a3 = t_ref[...].reshape(_EMBED, _VBLK // 128, 128)
    o_ref[...] = a3.transpose(1, 0, 2).reshape(_VBLK * _EMBED // 128, 128)


def _tc_detile(tableT):
    return pl.pallas_call(
        _detile_body,
        grid=(_NBLK,),
        in_specs=[pl.BlockSpec((_EMBED, _VBLK), lambda i: (0, i))],
        out_specs=pl.BlockSpec((_VBLK * _EMBED // 128, 128), lambda i: (i, 0)),
        out_shape=jax.ShapeDtypeStruct(
            (_NBLK * _VBLK * _EMBED // 128, 128), jnp.float32),
    )(tableT)


def kernel(x, table, W1, b1, W2, b2):
    x_chunks = x.astype(jnp.int32).reshape(_NW, _NCHUNK, _CHUNK)
    tab_flat = _tc_detile(table.T).reshape(_TLEN)
    w1r = W1.reshape(2 * _FEAT, 16)
    b2vec = jnp.full((16,), b2[0], jnp.float32)
    out = _fused(x_chunks, tab_flat, w1r, b1, W2.reshape(_HIDDEN), b2vec)
    return out.reshape(_B, 1)
